# baseline (speedup 1.0000x reference)
import functools
import math

import jax
import jax.numpy as jnp
from jax import lax
from jax.experimental import pallas as pl
from jax.experimental.pallas import tpu as pltpu

_V = 100000
_B = 1024
_S = 0.1 / (_V - 2)
_C = 0.9
_ENT = _C * math.log(_C) + (_V - 2) * _S * math.log(_S)

_NW = 4
_BR = 8
_ROWS_STEP = _NW * _BR
_NR = _B // _ROWS_STEP
_NFULL = _V // 128
_REM = _V - _NFULL * 128


def _stripe_partial(t, xw):
    mask = t != 0
    lane = lax.broadcasted_iota(jnp.int32, (_BR, 128), 1)
    ch0 = xw[:, 0:128]
    acc = ch0
    gacc = jnp.where(lane == t, ch0, 0.0)
    for c in range(1, _NFULL):
        ch = xw[:, c * 128:(c + 1) * 128]
        acc = acc + ch
        gacc = gacc + jnp.where(lane == t - c * 128, ch, 0.0)
    rs = jnp.sum(acc, axis=1, keepdims=True)
    gv = jnp.sum(gacc, axis=1, keepdims=True)
    if _REM:
        tch = xw[:, _NFULL * 128:_V]
        lane_t = lax.broadcasted_iota(jnp.int32, (_BR, _REM), 1)
        rs = rs + jnp.sum(tch, axis=1, keepdims=True)
        gv = gv + jnp.sum(
            jnp.where(lane_t == t - _NFULL * 128, tch, 0.0),
            axis=1, keepdims=True)
    x0 = xw[:, 0:1]
    per = jnp.where(mask, _ENT - (_C - _S) * gv - _S * (rs - x0), 0.0)
    return jnp.sum(per.astype(jnp.float32))


def _body(t_ref, x0_ref, x1_ref, x2_ref, x3_ref, o_ref):
    i = pl.program_id(0)
    partial = jnp.float32(0.0)
    for j, xw in enumerate((x0_ref, x1_ref, x2_ref, x3_ref)):
        tj = t_ref[j * _BR:(j + 1) * _BR, :]
        partial = partial + _stripe_partial(tj, xw)

    @pl.when(i == 0)
    def _init():
        o_ref[0, 0] = partial

    @pl.when(i > 0)
    def _acc():
        o_ref[0, 0] += partial


def _x_spec(j):
    return pl.BlockSpec((_BR, _V), lambda i, j=j: (_NW * i + j, 0))


def _tc_all(t2, x):
    out = pl.pallas_call(
        _body,
        grid=(_NR,),
        in_specs=[pl.BlockSpec((_ROWS_STEP, 1), lambda i: (i, 0))]
        + [_x_spec(j) for j in range(_NW)],
        out_specs=pl.BlockSpec(memory_space=pltpu.SMEM),
        out_shape=jax.ShapeDtypeStruct((1, 1), jnp.float32),
        compiler_params=pltpu.CompilerParams(
            dimension_semantics=("arbitrary",),
        ),
    )(t2, x, x, x, x)
    return out[0, 0]


@jax.jit
def kernel(x, target):
    return _tc_all(target.astype(jnp.int32).reshape(_B, 1), x)


# speedup vs baseline: 1.0425x; 1.0425x over previous
import functools
import math

import jax
import jax.numpy as jnp
from jax import lax
from jax.experimental import pallas as pl
from jax.experimental.pallas import tpu as pltpu

_V = 100000
_B = 1024
_S = 0.1 / (_V - 2)
_C = 0.9
_ENT = _C * math.log(_C) + (_V - 2) * _S * math.log(_S)

_NBUF = 4
_BR = 8
_ROWS_STEP = _NBUF * _BR
_NR = _B // _ROWS_STEP
_NSTRIPE = _B // _BR
_NFULL = _V // 128
_REM = _V - _NFULL * 128


def _stripe_partial(t, xw):
    mask = t != 0
    lane = lax.broadcasted_iota(jnp.int32, (_BR, 128), 1)
    ch0 = xw[:, 0:128]
    acc = ch0
    gacc = jnp.where(lane == t, ch0, 0.0)
    for c in range(1, _NFULL):
        ch = xw[:, c * 128:(c + 1) * 128]
        acc = acc + ch
        gacc = gacc + jnp.where(lane == t - c * 128, ch, 0.0)
    rs = jnp.sum(acc, axis=1, keepdims=True)
    gv = jnp.sum(gacc, axis=1, keepdims=True)
    if _REM:
        tch = xw[:, _NFULL * 128:_V]
        lane_t = lax.broadcasted_iota(jnp.int32, (_BR, _REM), 1)
        rs = rs + jnp.sum(tch, axis=1, keepdims=True)
        gv = gv + jnp.sum(
            jnp.where(lane_t == t - _NFULL * 128, tch, 0.0),
            axis=1, keepdims=True)
    x0 = xw[:, 0:1]
    per = jnp.where(mask, _ENT - (_C - _S) * gv - _S * (rs - x0), 0.0)
    return jnp.sum(per.astype(jnp.float32))


def _copy_stripe(x_hbm, bufs, sems, stripe, slot):
    return pltpu.make_async_copy(
        x_hbm.at[pl.ds(stripe * _BR, _BR), :], bufs.at[slot], sems.at[slot])


def _body(t_ref, x_hbm, o_ref, bufs, sems):
    i = pl.program_id(0)

    @pl.when(i == 0)
    def _prologue():
        for j in range(_NBUF):
            _copy_stripe(x_hbm, bufs, sems, j, j).start()

    partial = jnp.float32(0.0)
    for j in range(_NBUF):
        stripe = i * _NBUF + j
        _copy_stripe(x_hbm, bufs, sems, stripe, j).wait()
        partial = partial + _stripe_partial(
            t_ref[j * _BR:(j + 1) * _BR, :], bufs.at[j])

        @pl.when(stripe + _NBUF < _NSTRIPE)
        def _prefetch():
            _copy_stripe(x_hbm, bufs, sems, stripe + _NBUF, j).start()

    @pl.when(i == 0)
    def _init():
        o_ref[0, 0] = partial

    @pl.when(i > 0)
    def _acc():
        o_ref[0, 0] += partial


def _tc_all(t2, x):
    out = pl.pallas_call(
        _body,
        grid=(_NR,),
        in_specs=[
            pl.BlockSpec((_ROWS_STEP, 1), lambda i: (i, 0)),
            pl.BlockSpec(memory_space=pl.ANY),
        ],
        out_specs=pl.BlockSpec(memory_space=pltpu.SMEM),
        out_shape=jax.ShapeDtypeStruct((1, 1), jnp.float32),
        scratch_shapes=[
            pltpu.VMEM((_NBUF, _BR, _V), jnp.float32),
            pltpu.SemaphoreType.DMA((_NBUF,)),
        ],
        compiler_params=pltpu.CompilerParams(
            dimension_semantics=("arbitrary",),
        ),
    )(t2, x)
    return out[0, 0]


@jax.jit
def kernel(x, target):
    return _tc_all(target.astype(jnp.int32).reshape(_B, 1), x)
